# Initial kernel scaffold; baseline (speedup 1.0000x reference)
import jax
import jax.numpy as jnp
from jax.experimental import pallas as pl


def kernel(A_old, delta_A_episode):
    raise NotImplementedError("write your pallas kernel here")


# speedup vs baseline: 4019.1124x; 4019.1124x over previous
import functools

import jax
import jax.numpy as jnp
from jax.experimental import pallas as pl

ETA_C = 0.05
RADIUS = 0.693
D = 16
SB = 8
LB = 128


def _mm(x, y):
    acc = x[:, 0, None, :, :] * y[None, 0, :, :, :]
    for k in range(1, D):
        acc = acc + x[:, k, None, :, :] * y[None, k, :, :, :]
    return acc


def _transpose_m(x):
    return jnp.transpose(x, (1, 0, 2, 3))


def _lam_max(g, iters):
    v = jnp.ones((D, SB, LB), dtype=g.dtype)
    for _ in range(iters):
        w = jnp.sum(g * v[None, :, :, :], axis=1)
        nrm2 = jnp.sum(w * w, axis=0)
        v = w * jax.lax.rsqrt(nrm2 + 1e-30)[None, :, :]
    w = jnp.sum(g * v[None, :, :, :], axis=1)
    lam = jnp.sum(v * w, axis=0)
    return jnp.maximum(lam, 0.0)


def _body(a_ref, d_ref, o_ref):
    a = a_ref[...]
    de = d_ref[...]

    da = 0.5 * (de - _transpose_m(de))

    g_old = _mm(_transpose_m(a), a)
    norm_old = jnp.sqrt(_lam_max(g_old, 14))

    g_del = -_mm(da, da)
    norm_del = ETA_C * jnp.sqrt(_lam_max(g_del, 14))

    avail = jnp.maximum(RADIUS - norm_old, 1e-8)
    scale = jnp.minimum(avail / (norm_del + 1e-8), 1.0)
    das = da * scale[None, None, :, :]

    bracket = _mm(a, das) - _mm(das, a)
    a_new = a + ETA_C * das + (0.5 * ETA_C) * bracket

    ans = 0.5 * (a_new - _transpose_m(a_new))

    g_fin = -_mm(ans, ans)
    fin = jnp.sqrt(_lam_max(g_fin, 22))
    lim = jnp.minimum(RADIUS / (fin + 1e-8), 1.0)

    o_ref[...] = ans * lim[None, None, :, :]


@functools.partial(jax.jit, static_argnames=())
def kernel(A_old, delta_A_episode):
    B = A_old.shape[0]
    blk = SB * LB
    nb = B // blk
    at = jnp.transpose(A_old, (1, 2, 0)).reshape(D, D, nb * SB, LB)
    dt = jnp.transpose(delta_A_episode, (1, 2, 0)).reshape(D, D, nb * SB, LB)
    out = pl.pallas_call(
        _body,
        grid=(nb,),
        in_specs=[
            pl.BlockSpec((D, D, SB, LB), lambda i: (0, 0, i, 0)),
            pl.BlockSpec((D, D, SB, LB), lambda i: (0, 0, i, 0)),
        ],
        out_specs=pl.BlockSpec((D, D, SB, LB), lambda i: (0, 0, i, 0)),
        out_shape=jax.ShapeDtypeStruct((D, D, nb * SB, LB), jnp.float32),
    )(at, dt)
    return jnp.transpose(out.reshape(D, D, B), (2, 0, 1))


# speedup vs baseline: 8102.1625x; 2.0159x over previous
import functools

import jax
import jax.numpy as jnp
from jax.experimental import pallas as pl

ETA_C = 0.05
RADIUS = 0.693
D = 16
SB = 8
LB = 128


def _stack2(rows):
    return jnp.stack([jnp.stack(r) for r in rows])


def _skew_m(x):
    z = jnp.zeros_like(x[0, 0])
    s = [[None] * D for _ in range(D)]
    for i in range(D):
        s[i][i] = z
        for j in range(i + 1, D):
            e = 0.5 * (x[i, j] - x[j, i])
            s[i][j] = e
            s[j][i] = -e
    return _stack2(s)


def _gram_cols(x):
    g = [[None] * D for _ in range(D)]
    for i in range(D):
        for j in range(i, D):
            e = jnp.sum(x[:, i] * x[:, j], axis=0)
            g[i][j] = e
            g[j][i] = e
    return _stack2(g)


def _gram_rows(x):
    g = [[None] * D for _ in range(D)]
    for i in range(D):
        for j in range(i, D):
            e = jnp.sum(x[i] * x[j], axis=0)
            g[i][j] = e
            g[j][i] = e
    return _stack2(g)


def _bracket_skew(k_, d_):
    z = jnp.zeros_like(k_[0, 0])
    b = [[None] * D for _ in range(D)]
    for i in range(D):
        b[i][i] = z
        for j in range(i + 1, D):
            e = jnp.sum(d_[i] * k_[j] - k_[i] * d_[j], axis=0)
            b[i][j] = e
            b[j][i] = -e
    return _stack2(b)


def _lam_max(g, iters):
    v = jnp.ones((D, SB, LB), dtype=g.dtype)
    for _ in range(iters):
        w = jnp.sum(g * v[None, :, :, :], axis=1)
        nrm2 = jnp.sum(w * w, axis=0)
        v = w * jax.lax.rsqrt(nrm2 + 1e-30)[None, :, :]
    w = jnp.sum(g * v[None, :, :, :], axis=1)
    lam = jnp.sum(v * w, axis=0)
    return jnp.maximum(lam, 0.0)


def _body(a_ref, d_ref, o_ref):
    a = a_ref[...]
    de = d_ref[...]

    da = _skew_m(de)

    norm_old = jnp.sqrt(_lam_max(_gram_cols(a), 8))

    norm_del = ETA_C * jnp.sqrt(_lam_max(_gram_rows(da), 8))

    avail = jnp.maximum(RADIUS - norm_old, 1e-8)
    scale = jnp.minimum(avail / (norm_del + 1e-8), 1.0)
    das = da * scale[None, None, :, :]

    k_ = _skew_m(a)
    ans = k_ + ETA_C * das + (0.5 * ETA_C) * _bracket_skew(k_, das)

    fin = jnp.sqrt(_lam_max(_gram_rows(ans), 18))
    lim = jnp.minimum(RADIUS / (fin + 1e-8), 1.0)

    o_ref[...] = ans * lim[None, None, :, :]


@functools.partial(jax.jit, static_argnames=())
def kernel(A_old, delta_A_episode):
    B = A_old.shape[0]
    blk = SB * LB
    nb = B // blk
    at = jnp.transpose(A_old, (1, 2, 0)).reshape(D, D, nb * SB, LB)
    dt = jnp.transpose(delta_A_episode, (1, 2, 0)).reshape(D, D, nb * SB, LB)
    out = pl.pallas_call(
        _body,
        grid=(nb,),
        in_specs=[
            pl.BlockSpec((D, D, SB, LB), lambda i: (0, 0, i, 0)),
            pl.BlockSpec((D, D, SB, LB), lambda i: (0, 0, i, 0)),
        ],
        out_specs=pl.BlockSpec((D, D, SB, LB), lambda i: (0, 0, i, 0)),
        out_shape=jax.ShapeDtypeStruct((D, D, nb * SB, LB), jnp.float32),
    )(at, dt)
    return jnp.transpose(out.reshape(D, D, B), (2, 0, 1))


# speedup vs baseline: 9976.1471x; 1.2313x over previous
import jax
import jax.numpy as jnp
from jax.experimental import pallas as pl

ETA_C = 0.05
RADIUS = 0.693
D = 16
SB = 8
LB = 128
IT_OLD = 5
IT_DEL = 5
IT_FIN = 18


def _matvec(m, v):
    out = []
    for i in range(D):
        acc = None
        for j in range(D):
            if m[i][j] is None:
                continue
            t = m[i][j] * v[j]
            acc = t if acc is None else acc + t
        out.append(acc)
    return out


def _lam_max(g, iters):
    v = [jnp.ones((SB, LB), jnp.float32)] * D
    for _ in range(iters):
        w = _matvec(g, v)
        nrm2 = w[0] * w[0]
        for i in range(1, D):
            nrm2 = nrm2 + w[i] * w[i]
        r = jax.lax.rsqrt(nrm2 + 1e-30)
        v = [wi * r for wi in w]
    w = _matvec(g, v)
    lam = v[0] * w[0]
    for i in range(1, D):
        lam = lam + v[i] * w[i]
    return jnp.maximum(lam, 0.0)


def _gram(rows_a, rows_b):
    g = [[None] * D for _ in range(D)]
    for i in range(D):
        for j in range(i, D):
            acc = None
            for k in range(D):
                ra, rb = rows_a[i][k], rows_b[j][k]
                if ra is None or rb is None:
                    continue
                t = ra * rb
                acc = t if acc is None else acc + t
            g[i][j] = acc
            g[j][i] = acc
    return g


def _skew_pages(x):
    s = [[None] * D for _ in range(D)]
    for i in range(D):
        for j in range(i + 1, D):
            e = 0.5 * (x[i, j] - x[j, i])
            s[i][j] = e
            s[j][i] = -e
    return s


def _body(a_ref, d_ref, o_ref):
    a = a_ref[...]
    de = d_ref[...]

    da = _skew_pages(de)
    k_ = _skew_pages(a)

    a_cols = [[a[k, i] for k in range(D)] for i in range(D)]
    norm_old = jnp.sqrt(_lam_max(_gram(a_cols, a_cols), IT_OLD))

    norm_del = ETA_C * jnp.sqrt(_lam_max(_gram(da, da), IT_DEL))

    avail = jnp.maximum(RADIUS - norm_old, 1e-8)
    scale = jnp.minimum(avail / (norm_del + 1e-8), 1.0)

    das = [[None] * D for _ in range(D)]
    for i in range(D):
        for j in range(i + 1, D):
            e = da[i][j] * scale
            das[i][j] = e
            das[j][i] = -e

    he = 0.5 * ETA_C
    ans = [[None] * D for _ in range(D)]
    for i in range(D):
        for j in range(i + 1, D):
            br = None
            for t in range(D):
                if t == i or t == j:
                    continue
                u = das[i][t] * k_[j][t] - k_[i][t] * das[j][t]
                br = u if br is None else br + u
            e = k_[i][j] + ETA_C * das[i][j] + he * br
            ans[i][j] = e
            ans[j][i] = -e

    fin = jnp.sqrt(_lam_max(_gram(ans, ans), IT_FIN))
    lim = jnp.minimum(RADIUS / (fin + 1e-8), 1.0)

    z = jnp.zeros((SB, LB), jnp.float32)
    rows = []
    for i in range(D):
        rows.append(jnp.stack([z if ans[i][j] is None else ans[i][j] * lim
                               for j in range(D)]))
    o_ref[...] = jnp.stack(rows)


def kernel(A_old, delta_A_episode):
    B = A_old.shape[0]
    blk = SB * LB
    nb = B // blk
    at = jnp.transpose(A_old, (1, 2, 0)).reshape(D, D, nb * SB, LB)
    dt = jnp.transpose(delta_A_episode, (1, 2, 0)).reshape(D, D, nb * SB, LB)
    out = pl.pallas_call(
        _body,
        grid=(nb,),
        in_specs=[
            pl.BlockSpec((D, D, SB, LB), lambda i: (0, 0, i, 0)),
            pl.BlockSpec((D, D, SB, LB), lambda i: (0, 0, i, 0)),
        ],
        out_specs=pl.BlockSpec((D, D, SB, LB), lambda i: (0, 0, i, 0)),
        out_shape=jax.ShapeDtypeStruct((D, D, nb * SB, LB), jnp.float32),
    )(at, dt)
    return jnp.transpose(out.reshape(D, D, B), (2, 0, 1))
